# baseline (speedup 1.0000x reference)
import numpy as np
import jax
import jax.numpy as jnp
from jax import lax
from jax.experimental import pallas as pl
from jax.experimental.pallas import tpu as pltpu

_MIN_KEPT = 100000
_THRESH_BITS = int(np.float32(0.8).view(np.int32))

_HT = 256


_RC = 16


def _radix4_rounds(load_bits, nrows, u, shifts):
    kf = jnp.float32(_MIN_KEPT)
    for sh in shifts:
        q = jnp.int32(1 << sh)
        cands = (u + q, u + 2 * q, u + 3 * q)
        accs = [None, None, None]
        for r in range(nrows // _RC):
            blk = load_bits(r)
            for j in range(3):
                ind = jnp.where(blk < cands[j], 1.0, 0.0)
                accs[j] = ind if accs[j] is None else accs[j] + ind
        d = jnp.int32(0)
        for j in range(3):
            d = d + (jnp.sum(accs[j]) <= kf).astype(jnp.int32)
        u = u + d * q
    return u


def _ohem_body(score_ref, target_ref, out_ref, pred_s, loss_s, u_s):
    b = pl.program_id(0)
    t = pl.program_id(1)
    nb = pl.num_programs(0) - 1
    nc = score_ref.shape[1]

    @pl.when(jnp.logical_and(b == 0, t == 0))
    def _init():
        out_ref[0, 0] = 0.0

    @pl.when(b < nb)
    def _dense():
        slot = b % 2
        tgt = target_ref[0]
        x0 = score_ref[0, 0]
        s = jnp.exp(x0)
        x_t = jnp.where(tgt == 0, x0, 0.0)
        for c in range(1, nc):
            x_c = score_ref[0, c]
            s = s + jnp.exp(x_c)
            x_t = x_t + jnp.where(tgt == c, x_c, 0.0)
        pred_s[slot, pl.ds(t * _HT, _HT), :] = jnp.exp(x_t) / s
        loss_s[slot, pl.ds(t * _HT, _HT), :] = jnp.log(s) - x_t

    @pl.when(b >= 1)
    def _select():
        prev = (b + 1) % 2
        h = pred_s.shape[1]

        def load_bits(r):
            return lax.bitcast_convert_type(
                pred_s[prev, pl.ds(r * _RC, _RC), :], jnp.int32)

        @pl.when(t == 0)
        def _first_half():
            u_s[0] = _radix4_rounds(load_bits, h, jnp.int32(0),
                                    range(28, 13, -2))

        @pl.when(t == 1)
        def _second_half():
            u = _radix4_rounds(load_bits, h, u_s[0], range(12, -1, -2))
            thr = jnp.maximum(u, jnp.int32(_THRESH_BITS))
            accc = None
            accs = None
            for r in range(h // _RC):
                keep = load_bits(r) < thr
                ic = jnp.where(keep, 1.0, 0.0)
                il = jnp.where(keep, loss_s[prev, pl.ds(r * _RC, _RC), :],
                               0.0)
                accc = ic if accc is None else accc + ic
                accs = il if accs is None else accs + il
            cnt = jnp.sum(accc)
            sm = jnp.sum(accs)
            out_ref[0, 0] += sm / jnp.maximum(cnt, 1.0) / nb


@jax.jit
def kernel(score, target):
    batch, c, h, w = score.shape
    target = target.astype(jnp.int32)
    nht = h // _HT
    last = batch - 1
    out = pl.pallas_call(
        _ohem_body,
        grid=(batch + 1, nht),
        in_specs=[
            pl.BlockSpec((1, c, _HT, w),
                         lambda b, t: (jnp.minimum(b, last), 0,
                                       jnp.where(b > last, nht - 1, t), 0)),
            pl.BlockSpec((1, _HT, w),
                         lambda b, t: (jnp.minimum(b, last),
                                       jnp.where(b > last, nht - 1, t), 0)),
        ],
        out_specs=pl.BlockSpec(memory_space=pltpu.MemorySpace.SMEM),
        out_shape=jax.ShapeDtypeStruct((1, 1), jnp.float32),
        scratch_shapes=[
            pltpu.VMEM((2, h, w), jnp.float32),
            pltpu.VMEM((2, h, w), jnp.float32),
            pltpu.SMEM((1,), jnp.int32),
        ],
        compiler_params=pltpu.CompilerParams(
            dimension_semantics=("arbitrary", "arbitrary"),
        ),
    )(score, target)
    return out[0, 0]


# speedup vs baseline: 1.0019x; 1.0019x over previous
import numpy as np
import jax
import jax.numpy as jnp
from jax import lax
from jax.experimental import pallas as pl
from jax.experimental.pallas import tpu as pltpu

_MIN_KEPT = 100000
_THRESH_BITS = int(np.float32(0.8).view(np.int32))

_HT = 256


_RC = 32


def _radix4_rounds(load_bits, nrows, u, shifts):
    kf = jnp.float32(_MIN_KEPT)
    for sh in shifts:
        q = jnp.int32(1 << sh)
        cands = (u + q, u + 2 * q, u + 3 * q)
        accs = [None, None, None]
        for r in range(nrows // _RC):
            blk = load_bits(r)
            for j in range(3):
                ind = jnp.where(blk < cands[j], 1.0, 0.0)
                accs[j] = ind if accs[j] is None else accs[j] + ind
        d = jnp.int32(0)
        for j in range(3):
            d = d + (jnp.sum(accs[j]) <= kf).astype(jnp.int32)
        u = u + d * q
    return u


def _ohem_body(score_ref, target_ref, out_ref, pred_s, loss_s, u_s):
    b = pl.program_id(0)
    t = pl.program_id(1)
    nb = pl.num_programs(0) - 1
    nc = score_ref.shape[1]

    @pl.when(jnp.logical_and(b == 0, t == 0))
    def _init():
        out_ref[0, 0] = 0.0

    @pl.when(b < nb)
    def _dense():
        slot = b % 2
        tgt = target_ref[0]
        x0 = score_ref[0, 0]
        s = jnp.exp(x0)
        x_t = jnp.where(tgt == 0, x0, 0.0)
        for c in range(1, nc):
            x_c = score_ref[0, c]
            s = s + jnp.exp(x_c)
            x_t = x_t + jnp.where(tgt == c, x_c, 0.0)
        pred_s[slot, pl.ds(t * _HT, _HT), :] = jnp.exp(x_t) / s
        loss_s[slot, pl.ds(t * _HT, _HT), :] = jnp.log(s) - x_t

    @pl.when(b >= 1)
    def _select():
        prev = (b + 1) % 2
        h = pred_s.shape[1]

        def load_bits(r):
            return lax.bitcast_convert_type(
                pred_s[prev, pl.ds(r * _RC, _RC), :], jnp.int32)

        @pl.when(t == 0)
        def _first_half():
            u_s[0] = _radix4_rounds(load_bits, h, jnp.int32(0),
                                    range(28, 9, -2))

        @pl.when(t == 1)
        def _second_half():
            u = _radix4_rounds(load_bits, h, u_s[0], range(8, -1, -2))
            thr = jnp.maximum(u, jnp.int32(_THRESH_BITS))
            accc = None
            accs = None
            for r in range(h // _RC):
                keep = load_bits(r) < thr
                ic = jnp.where(keep, 1.0, 0.0)
                il = jnp.where(keep, loss_s[prev, pl.ds(r * _RC, _RC), :],
                               0.0)
                accc = ic if accc is None else accc + ic
                accs = il if accs is None else accs + il
            cnt = jnp.sum(accc)
            sm = jnp.sum(accs)
            out_ref[0, 0] += sm / jnp.maximum(cnt, 1.0) / nb


@jax.jit
def kernel(score, target):
    batch, c, h, w = score.shape
    target = target.astype(jnp.int32)
    nht = h // _HT
    last = batch - 1
    out = pl.pallas_call(
        _ohem_body,
        grid=(batch + 1, nht),
        in_specs=[
            pl.BlockSpec((1, c, _HT, w),
                         lambda b, t: (jnp.minimum(b, last), 0,
                                       jnp.where(b > last, nht - 1, t), 0)),
            pl.BlockSpec((1, _HT, w),
                         lambda b, t: (jnp.minimum(b, last),
                                       jnp.where(b > last, nht - 1, t), 0)),
        ],
        out_specs=pl.BlockSpec(memory_space=pltpu.MemorySpace.SMEM),
        out_shape=jax.ShapeDtypeStruct((1, 1), jnp.float32),
        scratch_shapes=[
            pltpu.VMEM((2, h, w), jnp.float32),
            pltpu.VMEM((2, h, w), jnp.float32),
            pltpu.SMEM((1,), jnp.int32),
        ],
        compiler_params=pltpu.CompilerParams(
            dimension_semantics=("arbitrary", "arbitrary"),
        ),
    )(score, target)
    return out[0, 0]
